# Initial kernel scaffold; baseline (speedup 1.0000x reference)
import functools

import jax
import jax.numpy as jnp
from jax import lax
from jax.experimental import pallas as pl
from jax.experimental.pallas import tpu as pltpu
from jax.experimental.pallas import tpu_sc as plsc

N = 10000
D = 256
DH = 128
EPS = 1e-5

NC = 2
NS = 16
K = 128

NPAD = 10240
ROWS_PER_TILE = NPAD // NS
ROW_BLOCKS = ROWS_PER_TILE // K

BM = 1000


def _build_indices(edges):
    e = edges.shape[0]
    i = edges[:, 0]
    j = edges[:, 1]
    dst = jnp.concatenate([i, j])
    src = jnp.concatenate([j, i])
    total = 2 * e
    c = -(-total // (NS * K))
    padded = NS * c * K
    pad = padded - total
    dst = jnp.concatenate([dst, jnp.full((pad,), N + 8, jnp.int32)])
    src = jnp.concatenate([src, jnp.zeros((pad,), jnp.int32)])
    return src.reshape(NS, c, K), dst.reshape(NS, c, K)


def _sc_aggregate(nbr_lo, nbr_hi, idx_src, idx_dst):
    c = idx_src.shape[1]
    mesh = plsc.VectorSubcoreMesh(core_axis_name="c", subcore_axis_name="s")

    @functools.partial(
        pl.kernel,
        out_type=jax.ShapeDtypeStruct((NC, NPAD, DH), jnp.float32),
        mesh=mesh,
        scratch_types=[
            pltpu.VMEM_SHARED((NPAD, DH), jnp.float32),
            pltpu.VMEM((c, K), jnp.int32),
            pltpu.VMEM((c, K), jnp.int32),
            pltpu.VMEM((K, DH), jnp.float32),
        ],
    )
    def k(lo_hbm, hi_hbm, isrc_hbm, idst_hbm, agg_hbm, acc, isrc_v, idst_v, gbuf):
        cid = lax.axis_index("c")
        sid = lax.axis_index("s")

        pltpu.sync_copy(isrc_hbm.at[sid], isrc_v)
        pltpu.sync_copy(idst_hbm.at[sid], idst_v)

        @pl.loop(0, K)
        def _(r):
            @pl.loop(0, DH, step=16)
            def _(c0):
                gbuf[r, pl.ds(c0, 16)] = jnp.zeros((16,), jnp.float32)

        @pl.loop(0, ROW_BLOCKS)
        def _(b):
            pltpu.sync_copy(gbuf, acc.at[pl.ds(sid * ROWS_PER_TILE + b * K, K)])

        plsc.subcore_barrier()

        def run(nbr_hbm):
            @pl.loop(0, c)
            def _(cc):
                pltpu.sync_copy(nbr_hbm.at[isrc_v.at[cc]], gbuf)
                pltpu.sync_copy(gbuf, acc.at[idst_v.at[cc]], add=True)

        @pl.when(cid == 0)
        def _():
            run(lo_hbm)

        @pl.when(cid == 1)
        def _():
            run(hi_hbm)

        plsc.subcore_barrier()

        @pl.loop(0, ROW_BLOCKS)
        def _(b):
            r0 = sid * ROWS_PER_TILE + b * K
            pltpu.sync_copy(acc.at[pl.ds(r0, K)], agg_hbm.at[cid, pl.ds(r0, K)])

    return k(nbr_lo, nbr_hi, idx_src, idx_dst)


def _tc_linear(x, w0, b0, w1, b1):
    def body(x_ref, w0_ref, b0_ref, w1_ref, b1_ref, out_ref, lo_ref, hi_ref):
        xb = x_ref[...]
        out_ref[...] = (
            jnp.dot(xb, w0_ref[...], preferred_element_type=jnp.float32)
            + b0_ref[...]
        )
        nb = (
            jnp.dot(xb, w1_ref[...], preferred_element_type=jnp.float32)
            + b1_ref[...]
        )
        lo_ref[...] = nb[:, :DH]
        hi_ref[...] = nb[:, DH:]

    grid = N // BM
    return pl.pallas_call(
        body,
        grid=(grid,),
        in_specs=[
            pl.BlockSpec((BM, D), lambda i: (i, 0)),
            pl.BlockSpec((D, D), lambda i: (0, 0)),
            pl.BlockSpec((1, D), lambda i: (0, 0)),
            pl.BlockSpec((D, D), lambda i: (0, 0)),
            pl.BlockSpec((1, D), lambda i: (0, 0)),
        ],
        out_specs=[
            pl.BlockSpec((BM, D), lambda i: (i, 0)),
            pl.BlockSpec((BM, DH), lambda i: (i, 0)),
            pl.BlockSpec((BM, DH), lambda i: (i, 0)),
        ],
        out_shape=[
            jax.ShapeDtypeStruct((N, D), jnp.float32),
            jax.ShapeDtypeStruct((N, DH), jnp.float32),
            jax.ShapeDtypeStruct((N, DH), jnp.float32),
        ],
    )(x, w0, b0.reshape(1, D), w1, b1.reshape(1, D))


def _tc_combine(out, agg2, g, be, res=None):
    def body(*refs):
        if res is None:
            out_ref, lo_ref, hi_ref, g_ref, be_ref, y_ref = refs
            r = 0.0
        else:
            out_ref, lo_ref, hi_ref, g_ref, be_ref, res_ref, y_ref = refs
            r = res_ref[...]
        y = out_ref[...] + jnp.concatenate([lo_ref[0], hi_ref[0]], axis=-1)
        mu = jnp.mean(y, axis=-1, keepdims=True)
        yc = y - mu
        var = jnp.mean(yc * yc, axis=-1, keepdims=True)
        yn = yc * lax.rsqrt(var + EPS) * g_ref[...] + be_ref[...]
        y_ref[...] = jnp.maximum(yn + r, 0.0)

    grid = N // BM
    in_specs = [
        pl.BlockSpec((BM, D), lambda i: (i, 0)),
        pl.BlockSpec((1, BM, DH), lambda i: (0, i, 0)),
        pl.BlockSpec((1, BM, DH), lambda i: (1, i, 0)),
        pl.BlockSpec((1, D), lambda i: (0, 0)),
        pl.BlockSpec((1, D), lambda i: (0, 0)),
    ]
    args = [out, agg2, agg2, g.reshape(1, D), be.reshape(1, D)]
    if res is not None:
        in_specs.append(pl.BlockSpec((BM, D), lambda i: (i, 0)))
        args.append(res)
    return pl.pallas_call(
        body,
        grid=(grid,),
        in_specs=in_specs,
        out_specs=pl.BlockSpec((BM, D), lambda i: (i, 0)),
        out_shape=jax.ShapeDtypeStruct((N, D), jnp.float32),
    )(*args)


def kernel(features, edges, w0_f, b0_f, w1_f, b1_f, g_f, be_f,
           w0_h1, b0_h1, w1_h1, b1_h1, g_h1, be_h1,
           w0_h2, b0_h2, w1_h2, b1_h2, g_h2, be_h2):
    idx_src, idx_dst = _build_indices(edges)
    layers = [
        (w0_f, b0_f, w1_f, b1_f, g_f, be_f),
        (w0_h1, b0_h1, w1_h1, b1_h1, g_h1, be_h1),
        (w0_h2, b0_h2, w1_h2, b1_h2, g_h2, be_h2),
    ]
    x = features
    for li, (w0, b0, w1, b1, g, be) in enumerate(layers):
        out, lo, hi = _tc_linear(x, w0, b0, w1, b1)
        agg2 = _sc_aggregate(lo, hi, idx_src, idx_dst)
        x = _tc_combine(out, agg2, g, be, res=features if li == 2 else None)
    return x


# speedup vs baseline: 2.1106x; 2.1106x over previous
import functools

import jax
import jax.numpy as jnp
from jax import lax
from jax.experimental import pallas as pl
from jax.experimental.pallas import tpu as pltpu
from jax.experimental.pallas import tpu_sc as plsc

N = 10000
D = 256
DH = 128
EPS = 1e-5

NC = 2
NS = 16
K = 128

NPAD = 10240
ROWS_PER_TILE = NPAD // NS
ROW_BLOCKS = ROWS_PER_TILE // K

BM = 1000


CB = 32


def _build_indices(edges):
    e = edges.shape[0]
    i = edges[:, 0]
    j = edges[:, 1]
    dst = jnp.concatenate([i, j])
    src = jnp.concatenate([j, i])
    total = 2 * e
    c = -(-total // (NS * K))
    c = -(-c // CB) * CB
    padded = NS * c * K
    pad = padded - total
    dst = jnp.concatenate([dst, jnp.full((pad,), N + 8, jnp.int32)])
    src = jnp.concatenate([src, jnp.zeros((pad,), jnp.int32)])
    return src.reshape(NS, c, K), dst.reshape(NS, c, K)


def _sc_aggregate(nbr_lo, nbr_hi, idx_src, idx_dst):
    c = idx_src.shape[1]
    mesh = plsc.VectorSubcoreMesh(core_axis_name="c", subcore_axis_name="s")

    @functools.partial(
        pl.kernel,
        out_type=jax.ShapeDtypeStruct((NC, NPAD, DH), jnp.float32),
        mesh=mesh,
        scratch_types=[
            pltpu.VMEM_SHARED((NPAD, DH), jnp.float32),
            pltpu.VMEM((CB, K), jnp.int32),
            pltpu.VMEM((CB, K), jnp.int32),
            pltpu.VMEM((K, DH), jnp.float32),
        ],
    )
    def k(lo_hbm, hi_hbm, isrc_hbm, idst_hbm, agg_hbm, acc, isrc_v, idst_v, gbuf):
        cid = lax.axis_index("c")
        sid = lax.axis_index("s")

        @pl.loop(0, K)
        def _(r):
            @pl.loop(0, DH, step=16)
            def _(c0):
                gbuf[r, pl.ds(c0, 16)] = jnp.zeros((16,), jnp.float32)

        @pl.loop(0, ROW_BLOCKS)
        def _(b):
            pltpu.sync_copy(gbuf, acc.at[pl.ds(sid * ROWS_PER_TILE + b * K, K)])

        plsc.subcore_barrier()

        def run(nbr_hbm):
            @pl.loop(0, c, step=CB)
            def _(c0):
                pltpu.sync_copy(isrc_hbm.at[sid, pl.ds(c0, CB)], isrc_v)
                pltpu.sync_copy(idst_hbm.at[sid, pl.ds(c0, CB)], idst_v)

                @pl.loop(0, CB)
                def _(cc):
                    pltpu.sync_copy(nbr_hbm.at[isrc_v.at[cc]], gbuf)
                    pltpu.sync_copy(gbuf, acc.at[idst_v.at[cc]], add=True)

        @pl.when(cid == 0)
        def _():
            run(lo_hbm)

        @pl.when(cid == 1)
        def _():
            run(hi_hbm)

        plsc.subcore_barrier()

        @pl.loop(0, ROW_BLOCKS)
        def _(b):
            r0 = sid * ROWS_PER_TILE + b * K
            pltpu.sync_copy(acc.at[pl.ds(r0, K)], agg_hbm.at[cid, pl.ds(r0, K)])

    return k(nbr_lo, nbr_hi, idx_src, idx_dst)


def _tc_linear(x, w0, b0, w1, b1):
    def body(x_ref, w0_ref, b0_ref, w1_ref, b1_ref, out_ref, lo_ref, hi_ref):
        xb = x_ref[...]
        out_ref[...] = (
            jnp.dot(xb, w0_ref[...], preferred_element_type=jnp.float32)
            + b0_ref[...]
        )
        nb = (
            jnp.dot(xb, w1_ref[...], preferred_element_type=jnp.float32)
            + b1_ref[...]
        )
        lo_ref[...] = nb[:, :DH]
        hi_ref[...] = nb[:, DH:]

    grid = N // BM
    return pl.pallas_call(
        body,
        grid=(grid,),
        in_specs=[
            pl.BlockSpec((BM, D), lambda i: (i, 0)),
            pl.BlockSpec((D, D), lambda i: (0, 0)),
            pl.BlockSpec((1, D), lambda i: (0, 0)),
            pl.BlockSpec((D, D), lambda i: (0, 0)),
            pl.BlockSpec((1, D), lambda i: (0, 0)),
        ],
        out_specs=[
            pl.BlockSpec((BM, D), lambda i: (i, 0)),
            pl.BlockSpec((BM, DH), lambda i: (i, 0)),
            pl.BlockSpec((BM, DH), lambda i: (i, 0)),
        ],
        out_shape=[
            jax.ShapeDtypeStruct((N, D), jnp.float32),
            jax.ShapeDtypeStruct((N, DH), jnp.float32),
            jax.ShapeDtypeStruct((N, DH), jnp.float32),
        ],
    )(x, w0, b0.reshape(1, D), w1, b1.reshape(1, D))


def _tc_combine(out, agg2, g, be, res=None):
    def body(*refs):
        if res is None:
            out_ref, lo_ref, hi_ref, g_ref, be_ref, y_ref = refs
            r = 0.0
        else:
            out_ref, lo_ref, hi_ref, g_ref, be_ref, res_ref, y_ref = refs
            r = res_ref[...]
        y = out_ref[...] + jnp.concatenate([lo_ref[0], hi_ref[0]], axis=-1)
        mu = jnp.mean(y, axis=-1, keepdims=True)
        yc = y - mu
        var = jnp.mean(yc * yc, axis=-1, keepdims=True)
        yn = yc * lax.rsqrt(var + EPS) * g_ref[...] + be_ref[...]
        y_ref[...] = jnp.maximum(yn + r, 0.0)

    grid = N // BM
    in_specs = [
        pl.BlockSpec((BM, D), lambda i: (i, 0)),
        pl.BlockSpec((1, BM, DH), lambda i: (0, i, 0)),
        pl.BlockSpec((1, BM, DH), lambda i: (1, i, 0)),
        pl.BlockSpec((1, D), lambda i: (0, 0)),
        pl.BlockSpec((1, D), lambda i: (0, 0)),
    ]
    args = [out, agg2, agg2, g.reshape(1, D), be.reshape(1, D)]
    if res is not None:
        in_specs.append(pl.BlockSpec((BM, D), lambda i: (i, 0)))
        args.append(res)
    return pl.pallas_call(
        body,
        grid=(grid,),
        in_specs=in_specs,
        out_specs=pl.BlockSpec((BM, D), lambda i: (i, 0)),
        out_shape=jax.ShapeDtypeStruct((N, D), jnp.float32),
    )(*args)


def kernel(features, edges, w0_f, b0_f, w1_f, b1_f, g_f, be_f,
           w0_h1, b0_h1, w1_h1, b1_h1, g_h1, be_h1,
           w0_h2, b0_h2, w1_h2, b1_h2, g_h2, be_h2):
    idx_src, idx_dst = _build_indices(edges)
    layers = [
        (w0_f, b0_f, w1_f, b1_f, g_f, be_f),
        (w0_h1, b0_h1, w1_h1, b1_h1, g_h1, be_h1),
        (w0_h2, b0_h2, w1_h2, b1_h2, g_h2, be_h2),
    ]
    x = features
    for li, (w0, b0, w1, b1, g, be) in enumerate(layers):
        out, lo, hi = _tc_linear(x, w0, b0, w1, b1)
        agg2 = _sc_aggregate(lo, hi, idx_src, idx_dst)
        x = _tc_combine(out, agg2, g, be, res=features if li == 2 else None)
    return x


# speedup vs baseline: 2.3473x; 1.1121x over previous
import functools

import jax
import jax.numpy as jnp
from jax import lax
from jax.experimental import pallas as pl
from jax.experimental.pallas import tpu as pltpu
from jax.experimental.pallas import tpu_sc as plsc

N = 10000
D = 256
DH = 128
EPS = 1e-5

NC = 2
NS = 16
K = 128

NPAD = 10240
ROWS_PER_TILE = NPAD // NS
ROW_BLOCKS = ROWS_PER_TILE // K

BM = 1000


CB = 40


def _build_indices(edges):
    e = edges.shape[0]
    i = edges[:, 0]
    j = edges[:, 1]
    dst = jnp.concatenate([i, j])
    src = jnp.concatenate([j, i])
    total = 2 * e
    c = -(-total // (NS * K))
    c = -(-c // CB) * CB
    padded = NS * c * K
    pad = padded - total
    dst = jnp.concatenate([dst, jnp.full((pad,), N + 8, jnp.int32)])
    src = jnp.concatenate([src, jnp.zeros((pad,), jnp.int32)])
    return src.reshape(NS, c, K), dst.reshape(NS, c, K)


def _sc_aggregate(nbr_lo, nbr_hi, idx_src, idx_dst):
    c = idx_src.shape[1]
    mesh = plsc.VectorSubcoreMesh(core_axis_name="c", subcore_axis_name="s")

    @functools.partial(
        pl.kernel,
        out_type=jax.ShapeDtypeStruct((NC, NPAD, DH), jnp.float32),
        mesh=mesh,
        scratch_types=[
            pltpu.VMEM_SHARED((NPAD, DH), jnp.float32),
            pltpu.VMEM((CB, K), jnp.int32),
            pltpu.VMEM((CB, K), jnp.int32),
            pltpu.VMEM((K, DH), jnp.float32),
            pltpu.VMEM((K, DH), jnp.float32),
            pltpu.SemaphoreType.DMA,
            pltpu.SemaphoreType.DMA,
            pltpu.SemaphoreType.DMA,
            pltpu.SemaphoreType.DMA,
        ],
    )
    def k(lo_hbm, hi_hbm, isrc_hbm, idst_hbm, agg_hbm, acc, isrc_v, idst_v,
          gbuf0, gbuf1, gsem0, gsem1, ssem0, ssem1):
        cid = lax.axis_index("c")
        sid = lax.axis_index("s")

        @pl.loop(0, K)
        def _(r):
            @pl.loop(0, DH, step=16)
            def _(c0):
                gbuf0[r, pl.ds(c0, 16)] = jnp.zeros((16,), jnp.float32)

        @pl.loop(0, ROW_BLOCKS)
        def _(b):
            pltpu.sync_copy(gbuf0, acc.at[pl.ds(sid * ROWS_PER_TILE + b * K, K)])

        plsc.subcore_barrier()

        def run(nbr_hbm):
            @pl.loop(0, c, step=CB)
            def _(c0):
                pltpu.sync_copy(isrc_hbm.at[sid, pl.ds(c0, CB)], isrc_v)
                pltpu.sync_copy(idst_hbm.at[sid, pl.ds(c0, CB)], idst_v)

                pltpu.async_copy(nbr_hbm.at[isrc_v.at[0]], gbuf0, gsem0)
                pltpu.async_copy(nbr_hbm.at[isrc_v.at[1]], gbuf1, gsem1)

                @pl.loop(0, CB, step=2)
                def _(cc):
                    pltpu.make_async_copy(
                        nbr_hbm.at[isrc_v.at[cc]], gbuf0, gsem0).wait()
                    pltpu.async_copy(
                        gbuf0, acc.at[idst_v.at[cc]], ssem0, add=True)

                    pltpu.make_async_copy(
                        nbr_hbm.at[isrc_v.at[cc + 1]], gbuf1, gsem1).wait()
                    pltpu.async_copy(
                        gbuf1, acc.at[idst_v.at[cc + 1]], ssem1, add=True)

                    @pl.when(cc + 2 < CB)
                    def _():
                        pltpu.make_async_copy(
                            gbuf0, acc.at[idst_v.at[cc]], ssem0).wait()
                        pltpu.async_copy(
                            nbr_hbm.at[isrc_v.at[cc + 2]], gbuf0, gsem0)

                        pltpu.make_async_copy(
                            gbuf1, acc.at[idst_v.at[cc + 1]], ssem1).wait()
                        pltpu.async_copy(
                            nbr_hbm.at[isrc_v.at[cc + 3]], gbuf1, gsem1)

                pltpu.make_async_copy(
                    gbuf0, acc.at[idst_v.at[CB - 2]], ssem0).wait()
                pltpu.make_async_copy(
                    gbuf1, acc.at[idst_v.at[CB - 1]], ssem1).wait()

        @pl.when(cid == 0)
        def _():
            run(lo_hbm)

        @pl.when(cid == 1)
        def _():
            run(hi_hbm)

        plsc.subcore_barrier()

        @pl.loop(0, ROW_BLOCKS)
        def _(b):
            r0 = sid * ROWS_PER_TILE + b * K
            pltpu.sync_copy(acc.at[pl.ds(r0, K)], agg_hbm.at[cid, pl.ds(r0, K)])

    return k(nbr_lo, nbr_hi, idx_src, idx_dst)


def _tc_linear(x, w0, b0, w1, b1):
    def body(x_ref, w0_ref, b0_ref, w1_ref, b1_ref, out_ref, lo_ref, hi_ref):
        xb = x_ref[...]
        out_ref[...] = (
            jnp.dot(xb, w0_ref[...], preferred_element_type=jnp.float32)
            + b0_ref[...]
        )
        nb = (
            jnp.dot(xb, w1_ref[...], preferred_element_type=jnp.float32)
            + b1_ref[...]
        )
        lo_ref[...] = nb[:, :DH]
        hi_ref[...] = nb[:, DH:]

    grid = N // BM
    return pl.pallas_call(
        body,
        grid=(grid,),
        in_specs=[
            pl.BlockSpec((BM, D), lambda i: (i, 0)),
            pl.BlockSpec((D, D), lambda i: (0, 0)),
            pl.BlockSpec((1, D), lambda i: (0, 0)),
            pl.BlockSpec((D, D), lambda i: (0, 0)),
            pl.BlockSpec((1, D), lambda i: (0, 0)),
        ],
        out_specs=[
            pl.BlockSpec((BM, D), lambda i: (i, 0)),
            pl.BlockSpec((BM, DH), lambda i: (i, 0)),
            pl.BlockSpec((BM, DH), lambda i: (i, 0)),
        ],
        out_shape=[
            jax.ShapeDtypeStruct((N, D), jnp.float32),
            jax.ShapeDtypeStruct((N, DH), jnp.float32),
            jax.ShapeDtypeStruct((N, DH), jnp.float32),
        ],
    )(x, w0, b0.reshape(1, D), w1, b1.reshape(1, D))


def _tc_combine(out, agg2, g, be, res=None):
    def body(*refs):
        if res is None:
            out_ref, lo_ref, hi_ref, g_ref, be_ref, y_ref = refs
            r = 0.0
        else:
            out_ref, lo_ref, hi_ref, g_ref, be_ref, res_ref, y_ref = refs
            r = res_ref[...]
        y = out_ref[...] + jnp.concatenate([lo_ref[0], hi_ref[0]], axis=-1)
        mu = jnp.mean(y, axis=-1, keepdims=True)
        yc = y - mu
        var = jnp.mean(yc * yc, axis=-1, keepdims=True)
        yn = yc * lax.rsqrt(var + EPS) * g_ref[...] + be_ref[...]
        y_ref[...] = jnp.maximum(yn + r, 0.0)

    grid = N // BM
    in_specs = [
        pl.BlockSpec((BM, D), lambda i: (i, 0)),
        pl.BlockSpec((1, BM, DH), lambda i: (0, i, 0)),
        pl.BlockSpec((1, BM, DH), lambda i: (1, i, 0)),
        pl.BlockSpec((1, D), lambda i: (0, 0)),
        pl.BlockSpec((1, D), lambda i: (0, 0)),
    ]
    args = [out, agg2, agg2, g.reshape(1, D), be.reshape(1, D)]
    if res is not None:
        in_specs.append(pl.BlockSpec((BM, D), lambda i: (i, 0)))
        args.append(res)
    return pl.pallas_call(
        body,
        grid=(grid,),
        in_specs=in_specs,
        out_specs=pl.BlockSpec((BM, D), lambda i: (i, 0)),
        out_shape=jax.ShapeDtypeStruct((N, D), jnp.float32),
    )(*args)


def kernel(features, edges, w0_f, b0_f, w1_f, b1_f, g_f, be_f,
           w0_h1, b0_h1, w1_h1, b1_h1, g_h1, be_h1,
           w0_h2, b0_h2, w1_h2, b1_h2, g_h2, be_h2):
    idx_src, idx_dst = _build_indices(edges)
    layers = [
        (w0_f, b0_f, w1_f, b1_f, g_f, be_f),
        (w0_h1, b0_h1, w1_h1, b1_h1, g_h1, be_h1),
        (w0_h2, b0_h2, w1_h2, b1_h2, g_h2, be_h2),
    ]
    x = features
    for li, (w0, b0, w1, b1, g, be) in enumerate(layers):
        out, lo, hi = _tc_linear(x, w0, b0, w1, b1)
        agg2 = _sc_aggregate(lo, hi, idx_src, idx_dst)
        x = _tc_combine(out, agg2, g, be, res=features if li == 2 else None)
    return x


# speedup vs baseline: 2.6129x; 1.1132x over previous
import functools

import jax
import jax.numpy as jnp
from jax import lax
from jax.experimental import pallas as pl
from jax.experimental.pallas import tpu as pltpu
from jax.experimental.pallas import tpu_sc as plsc

N = 10000
D = 256
DH = 128
EPS = 1e-5

NC = 2
NS = 16
K = 64
NBUF = 4

NPAD = 10240
ROWS_PER_TILE = NPAD // NS
ROW_BLOCKS = ROWS_PER_TILE // K

BM = 1000


CB = 40


def _build_indices(edges):
    e = edges.shape[0]
    i = edges[:, 0]
    j = edges[:, 1]
    dst = jnp.concatenate([i, j])
    src = jnp.concatenate([j, i])
    total = 2 * e
    c = -(-total // (NS * K))
    c = -(-c // CB) * CB
    padded = NS * c * K
    pad = padded - total
    dst = jnp.concatenate([dst, jnp.full((pad,), N + 8, jnp.int32)])
    src = jnp.concatenate([src, jnp.zeros((pad,), jnp.int32)])
    return src.reshape(NS, c, K), dst.reshape(NS, c, K)


def _sc_aggregate(nbr_lo, nbr_hi, idx_src, idx_dst):
    c = idx_src.shape[1]
    mesh = plsc.VectorSubcoreMesh(core_axis_name="c", subcore_axis_name="s")

    @functools.partial(
        pl.kernel,
        out_type=jax.ShapeDtypeStruct((NC, NPAD, DH), jnp.float32),
        mesh=mesh,
        scratch_types=(
            [pltpu.VMEM_SHARED((NPAD, DH), jnp.float32)]
            + [pltpu.VMEM((CB, K), jnp.int32)] * 2
            + [pltpu.VMEM((K, DH), jnp.float32)] * NBUF
            + [pltpu.SemaphoreType.DMA] * (2 * NBUF)
        ),
    )
    def k(lo_hbm, hi_hbm, isrc_hbm, idst_hbm, agg_hbm, acc, isrc_v, idst_v,
          *bufs_and_sems):
        gbuf = bufs_and_sems[:NBUF]
        gsem = bufs_and_sems[NBUF:2 * NBUF]
        ssem = bufs_and_sems[2 * NBUF:]
        cid = lax.axis_index("c")
        sid = lax.axis_index("s")

        @pl.loop(0, K)
        def _(r):
            @pl.loop(0, DH, step=16)
            def _(c0):
                gbuf[0][r, pl.ds(c0, 16)] = jnp.zeros((16,), jnp.float32)

        @pl.loop(0, ROW_BLOCKS)
        def _(b):
            pltpu.sync_copy(gbuf[0],
                            acc.at[pl.ds(sid * ROWS_PER_TILE + b * K, K)])

        plsc.subcore_barrier()

        def run(nbr_hbm):
            @pl.loop(0, c, step=CB)
            def _(c0):
                pltpu.sync_copy(isrc_hbm.at[sid, pl.ds(c0, CB)], isrc_v)
                pltpu.sync_copy(idst_hbm.at[sid, pl.ds(c0, CB)], idst_v)

                for b in range(NBUF):
                    pltpu.async_copy(
                        nbr_hbm.at[isrc_v.at[b]], gbuf[b], gsem[b])

                @pl.loop(0, CB, step=NBUF)
                def _(cc):
                    for b in range(NBUF):
                        pltpu.make_async_copy(
                            nbr_hbm.at[isrc_v.at[cc + b]],
                            gbuf[b], gsem[b]).wait()
                        pltpu.async_copy(
                            gbuf[b], acc.at[idst_v.at[cc + b]],
                            ssem[b], add=True)

                    for b in range(NBUF):
                        @pl.when(cc + NBUF + b < CB)
                        def _(b=b):
                            pltpu.make_async_copy(
                                gbuf[b], acc.at[idst_v.at[cc + b]],
                                ssem[b]).wait()
                            pltpu.async_copy(
                                nbr_hbm.at[isrc_v.at[cc + NBUF + b]],
                                gbuf[b], gsem[b])

                for b in range(NBUF):
                    pltpu.make_async_copy(
                        gbuf[b], acc.at[idst_v.at[CB - NBUF + b]],
                        ssem[b]).wait()

        @pl.when(cid == 0)
        def _():
            run(lo_hbm)

        @pl.when(cid == 1)
        def _():
            run(hi_hbm)

        plsc.subcore_barrier()

        @pl.loop(0, ROW_BLOCKS)
        def _(b):
            r0 = sid * ROWS_PER_TILE + b * K
            pltpu.sync_copy(acc.at[pl.ds(r0, K)], agg_hbm.at[cid, pl.ds(r0, K)])

    return k(nbr_lo, nbr_hi, idx_src, idx_dst)


def _tc_linear(x, w0, b0, w1, b1):
    def body(x_ref, w0_ref, b0_ref, w1_ref, b1_ref, out_ref, lo_ref, hi_ref):
        xb = x_ref[...]
        out_ref[...] = (
            jnp.dot(xb, w0_ref[...], preferred_element_type=jnp.float32)
            + b0_ref[...]
        )
        nb = (
            jnp.dot(xb, w1_ref[...], preferred_element_type=jnp.float32)
            + b1_ref[...]
        )
        lo_ref[...] = nb[:, :DH]
        hi_ref[...] = nb[:, DH:]

    grid = N // BM
    return pl.pallas_call(
        body,
        grid=(grid,),
        in_specs=[
            pl.BlockSpec((BM, D), lambda i: (i, 0)),
            pl.BlockSpec((D, D), lambda i: (0, 0)),
            pl.BlockSpec((1, D), lambda i: (0, 0)),
            pl.BlockSpec((D, D), lambda i: (0, 0)),
            pl.BlockSpec((1, D), lambda i: (0, 0)),
        ],
        out_specs=[
            pl.BlockSpec((BM, D), lambda i: (i, 0)),
            pl.BlockSpec((BM, DH), lambda i: (i, 0)),
            pl.BlockSpec((BM, DH), lambda i: (i, 0)),
        ],
        out_shape=[
            jax.ShapeDtypeStruct((N, D), jnp.float32),
            jax.ShapeDtypeStruct((N, DH), jnp.float32),
            jax.ShapeDtypeStruct((N, DH), jnp.float32),
        ],
    )(x, w0, b0.reshape(1, D), w1, b1.reshape(1, D))


def _tc_combine(out, agg2, g, be, res=None):
    def body(*refs):
        if res is None:
            out_ref, lo_ref, hi_ref, g_ref, be_ref, y_ref = refs
            r = 0.0
        else:
            out_ref, lo_ref, hi_ref, g_ref, be_ref, res_ref, y_ref = refs
            r = res_ref[...]
        y = out_ref[...] + jnp.concatenate([lo_ref[0], hi_ref[0]], axis=-1)
        mu = jnp.mean(y, axis=-1, keepdims=True)
        yc = y - mu
        var = jnp.mean(yc * yc, axis=-1, keepdims=True)
        yn = yc * lax.rsqrt(var + EPS) * g_ref[...] + be_ref[...]
        y_ref[...] = jnp.maximum(yn + r, 0.0)

    grid = N // BM
    in_specs = [
        pl.BlockSpec((BM, D), lambda i: (i, 0)),
        pl.BlockSpec((1, BM, DH), lambda i: (0, i, 0)),
        pl.BlockSpec((1, BM, DH), lambda i: (1, i, 0)),
        pl.BlockSpec((1, D), lambda i: (0, 0)),
        pl.BlockSpec((1, D), lambda i: (0, 0)),
    ]
    args = [out, agg2, agg2, g.reshape(1, D), be.reshape(1, D)]
    if res is not None:
        in_specs.append(pl.BlockSpec((BM, D), lambda i: (i, 0)))
        args.append(res)
    return pl.pallas_call(
        body,
        grid=(grid,),
        in_specs=in_specs,
        out_specs=pl.BlockSpec((BM, D), lambda i: (i, 0)),
        out_shape=jax.ShapeDtypeStruct((N, D), jnp.float32),
    )(*args)


def kernel(features, edges, w0_f, b0_f, w1_f, b1_f, g_f, be_f,
           w0_h1, b0_h1, w1_h1, b1_h1, g_h1, be_h1,
           w0_h2, b0_h2, w1_h2, b1_h2, g_h2, be_h2):
    idx_src, idx_dst = _build_indices(edges)
    layers = [
        (w0_f, b0_f, w1_f, b1_f, g_f, be_f),
        (w0_h1, b0_h1, w1_h1, b1_h1, g_h1, be_h1),
        (w0_h2, b0_h2, w1_h2, b1_h2, g_h2, be_h2),
    ]
    x = features
    for li, (w0, b0, w1, b1, g, be) in enumerate(layers):
        out, lo, hi = _tc_linear(x, w0, b0, w1, b1)
        agg2 = _sc_aggregate(lo, hi, idx_src, idx_dst)
        x = _tc_combine(out, agg2, g, be, res=features if li == 2 else None)
    return x
